# baseline (speedup 1.0000x reference)
import functools

import jax
import jax.numpy as jnp
from jax.experimental import pallas as pl
from jax.experimental.pallas import tpu as pltpu
from jax.experimental.pallas import tpu_sc as plsc

DIM = 768
NUM_EXPERTS = 64
CAPACITY_FACTOR = 1.0
EPSILON = 1e-06

BM = 4096
NC = 2
NS = 16
ZCHUNK = 256


def _stage_a(x_ref, w_ref, b_ref, score_ref, amax_ref, denom_ref):
    j = pl.program_id(0)
    xb = x_ref[0]
    logits = jax.lax.dot_general(
        w_ref[...], xb,
        (((1,), (1,)), ((), ())),
        preferred_element_type=jnp.float32,
    ) + b_ref[...]
    m = jnp.max(logits, axis=0, keepdims=True)
    idx = jax.lax.broadcasted_iota(jnp.int32, logits.shape, 0)
    a = jnp.min(jnp.where(logits == m, idx, NUM_EXPERTS), axis=0)
    s = jnp.sum(jnp.exp(logits - m), axis=0)
    score = 1.0 / s
    score_ref[0, 0, :] = score
    amax_ref[0, 0, :] = a
    onehot = (idx == a[None, :]).astype(jnp.float32)
    contrib = jax.lax.dot_general(
        onehot * score[None, :], jnp.ones((BM, 1), jnp.float32),
        (((1,), (0,)), ((), ())),
        preferred_element_type=jnp.float32,
    )

    @pl.when(j == 0)
    def _():
        denom_ref[...] = jnp.zeros_like(denom_ref)

    denom_ref[...] += contrib


def _make_sc_zero(total):
    words_per_tile = total // (NC * NS)
    zw = ZCHUNK * NUM_EXPERTS
    nchunk = words_per_tile // zw

    def body(out_hbm, zbuf, sem):
        wid = jax.lax.axis_index("s") * NC + jax.lax.axis_index("c")
        base = wid * words_per_tile
        zeros16 = jnp.zeros((16,), jnp.float32)

        def zb(i, carry):
            for u in range(8):
                zbuf[pl.ds(i * 128 + u * 16, 16)] = zeros16
            return carry

        jax.lax.fori_loop(0, zw // 128, zb, 0)
        copies = []
        for k in range(nchunk):
            copies.append(
                pltpu.async_copy(zbuf, out_hbm.at[pl.ds(base + k * zw, zw)], sem)
            )
        for cp in copies:
            cp.wait()

    return functools.partial(
        pl.kernel,
        mesh=plsc.VectorSubcoreMesh(core_axis_name="c", subcore_axis_name="s"),
        compiler_params=pltpu.CompilerParams(needs_layout_passes=False),
        out_type=jax.ShapeDtypeStruct((total,), jnp.float32),
        scratch_types=[
            pltpu.VMEM((ZCHUNK * NUM_EXPERTS,), jnp.float32),
            pltpu.SemaphoreType.DMA,
        ],
    )(body)


def _make_sc_scatter(batch, n_tok):
    rows = batch * n_tok
    rpt = rows // (NC * NS)
    tiles_per_b = n_tok // rpt
    ngrp = rpt // 128
    cap = float(rows * CAPACITY_FACTOR)

    def body(score_hbm, amax_hbm, denom_hbm, out_ref,
             score_v, amax_v, denom_v, inv_v, idx_v, val_v, sem, semi):
        wid = jax.lax.axis_index("s") * NC + jax.lax.axis_index("c")
        base = wid * rpt
        bidx = wid // tiles_per_b
        n0 = (wid % tiles_per_b) * rpt
        flat0 = bidx * NUM_EXPERTS * n_tok + n0
        pltpu.sync_copy(score_hbm.at[pl.ds(base, rpt)], score_v)
        pltpu.sync_copy(amax_hbm.at[pl.ds(base, rpt)], amax_v)
        pltpu.sync_copy(denom_hbm, denom_v)
        for t in range(NUM_EXPERTS // 16):
            d = denom_v[pl.ds(t * 16, 16)]
            inv_v[pl.ds(t * 16, 16)] = cap / (d + EPSILON)

        iota16 = jax.lax.iota(jnp.int32, 16)
        for g in range(ngrp):
            for u in range(8):
                rb = g * 128 + u * 16
                a16 = amax_v[pl.ds(rb, 16)]
                val_v[g, pl.ds(u * 16, 16)] = (
                    score_v[pl.ds(rb, 16)] * plsc.load_gather(inv_v, [a16])
                )
                idx_v[g, pl.ds(u * 16, 16)] = flat0 + rb + iota16 + a16 * n_tok
        copies = []
        for g in range(ngrp):
            copies.append(
                pltpu.async_copy(val_v.at[g], out_ref.at[idx_v.at[g]], sem)
            )
        for cp in copies:
            cp.wait()

    return functools.partial(
        pl.kernel,
        mesh=plsc.VectorSubcoreMesh(core_axis_name="c", subcore_axis_name="s"),
        compiler_params=pltpu.CompilerParams(needs_layout_passes=False),
        out_type=(),
        scratch_types=[
            pltpu.VMEM((rpt,), jnp.float32),
            pltpu.VMEM((rpt,), jnp.int32),
            pltpu.VMEM((NUM_EXPERTS,), jnp.float32),
            pltpu.VMEM((NUM_EXPERTS,), jnp.float32),
            pltpu.VMEM((rpt // 128, 128), jnp.int32),
            pltpu.VMEM((rpt // 128, 128), jnp.float32),
            pltpu.SemaphoreType.DMA,
            pltpu.SemaphoreType.DMA,
        ],
    )(body)


def kernel(x, W, b):
    batch, N, dim = x.shape
    rows = batch * N
    nb = rows // BM
    per_batch = N // BM
    b2 = b.reshape(NUM_EXPERTS, 1)

    out0 = _make_sc_zero(rows * NUM_EXPERTS)()

    score, amax, denom = pl.pallas_call(
        _stage_a,
        grid=(nb,),
        in_specs=[
            pl.BlockSpec((1, BM, dim), lambda j: (j // per_batch, j % per_batch, 0)),
            pl.BlockSpec((NUM_EXPERTS, dim), lambda j: (0, 0)),
            pl.BlockSpec((NUM_EXPERTS, 1), lambda j: (0, 0)),
        ],
        out_specs=[
            pl.BlockSpec((1, 1, BM), lambda j: (j, 0, 0)),
            pl.BlockSpec((1, 1, BM), lambda j: (j, 0, 0)),
            pl.BlockSpec((NUM_EXPERTS, 1), lambda j: (0, 0)),
        ],
        out_shape=[
            jax.ShapeDtypeStruct((nb, 1, BM), jnp.float32),
            jax.ShapeDtypeStruct((nb, 1, BM), jnp.int32),
            jax.ShapeDtypeStruct((NUM_EXPERTS, 1), jnp.float32),
        ],
    )(x, W, b2)

    out_ref = jax.new_ref(out0)
    scatter = _make_sc_scatter(batch, N)
    scatter(score.reshape(rows), amax.reshape(rows), denom.reshape(NUM_EXPERTS),
            out_ref)
    out_t = out_ref[...].reshape(batch, NUM_EXPERTS, N)
    return jnp.transpose(out_t, (0, 2, 1))


# speedup vs baseline: 1.3915x; 1.3915x over previous
import functools

import jax
import jax.numpy as jnp
from jax.experimental import pallas as pl
from jax.experimental.pallas import tpu as pltpu
from jax.experimental.pallas import tpu_sc as plsc

DIM = 768
NUM_EXPERTS = 64
CAPACITY_FACTOR = 1.0
EPSILON = 1e-06

BM = 2048
NC = 2
NS = 16


def _stage_a(x_ref, w_ref, b_ref, score_ref, amax_ref, denom_ref):
    j = pl.program_id(0)
    xb = x_ref[0]
    logits = jax.lax.dot_general(
        w_ref[...], xb,
        (((1,), (1,)), ((), ())),
        preferred_element_type=jnp.float32,
    ) + b_ref[...]
    m = jnp.max(logits, axis=0, keepdims=True)
    idx = jax.lax.broadcasted_iota(jnp.int32, logits.shape, 0)
    a = jnp.min(jnp.where(logits == m, idx, NUM_EXPERTS), axis=0)
    s = jnp.sum(jnp.exp(logits - m), axis=0)
    score = 1.0 / s
    score_ref[0, 0, :] = score
    amax_ref[0, 0, :] = a
    onehot = (idx == a[None, :]).astype(jnp.float32)
    contrib = jax.lax.dot_general(
        onehot * score[None, :], jnp.ones((BM, 1), jnp.float32),
        (((1,), (0,)), ((), ())),
        preferred_element_type=jnp.float32,
    )

    @pl.when(j == 0)
    def _():
        denom_ref[...] = jnp.zeros_like(denom_ref)

    denom_ref[...] += contrib


def _make_sc_expand(batch, n_tok):
    rows = batch * n_tok
    rpt = rows // (NC * NS)
    tiles_per_b = n_tok // rpt
    slab = NUM_EXPERTS * rpt
    cap = float(rows * CAPACITY_FACTOR)

    def body(score_hbm, amax_hbm, denom_hbm, out_ref,
             score_v, amax_v, denom_v, inv_v, buf, sem):
        wid = jax.lax.axis_index("s") * NC + jax.lax.axis_index("c")
        base = wid * rpt
        bidx = wid // tiles_per_b
        n0 = (wid % tiles_per_b) * rpt
        pltpu.sync_copy(score_hbm.at[pl.ds(base, rpt)], score_v)
        pltpu.sync_copy(amax_hbm.at[pl.ds(base, rpt)], amax_v)
        pltpu.sync_copy(denom_hbm, denom_v)
        for t in range(NUM_EXPERTS // 16):
            d = denom_v[pl.ds(t * 16, 16)]
            inv_v[pl.ds(t * 16, 16)] = cap / (d + EPSILON)

        zeros16 = jnp.zeros((16,), jnp.float32)

        def zb(i, carry):
            for u in range(8):
                buf[pl.ds(i * 128 + u * 16, 16)] = zeros16
            return carry

        jax.lax.fori_loop(0, slab // 128, zb, 0)

        iota16 = jax.lax.iota(jnp.int32, 16)
        for g in range(rpt // 16):
            rb = g * 16
            a16 = amax_v[pl.ds(rb, 16)]
            v16 = score_v[pl.ds(rb, 16)] * plsc.load_gather(inv_v, [a16])
            plsc.store_scatter(buf, [a16 * rpt + rb + iota16], v16)

        flat0 = bidx * NUM_EXPERTS * n_tok + n0
        copies = []
        for e in range(NUM_EXPERTS):
            copies.append(
                pltpu.async_copy(
                    buf.at[pl.ds(e * rpt, rpt)],
                    out_ref.at[pl.ds(flat0 + e * n_tok, rpt)],
                    sem,
                )
            )
        for cp in copies:
            cp.wait()

    return functools.partial(
        pl.kernel,
        mesh=plsc.VectorSubcoreMesh(core_axis_name="c", subcore_axis_name="s"),
        compiler_params=pltpu.CompilerParams(needs_layout_passes=False),
        out_type=jax.ShapeDtypeStruct((rows * NUM_EXPERTS,), jnp.float32),
        scratch_types=[
            pltpu.VMEM((rpt,), jnp.float32),
            pltpu.VMEM((rpt,), jnp.int32),
            pltpu.VMEM((NUM_EXPERTS,), jnp.float32),
            pltpu.VMEM((NUM_EXPERTS,), jnp.float32),
            pltpu.VMEM((slab,), jnp.float32),
            pltpu.SemaphoreType.DMA,
        ],
    )(body)


def kernel(x, W, b):
    batch, N, dim = x.shape
    rows = batch * N
    nb = rows // BM
    per_batch = N // BM
    b2 = b.reshape(NUM_EXPERTS, 1)

    score, amax, denom = pl.pallas_call(
        _stage_a,
        grid=(nb,),
        in_specs=[
            pl.BlockSpec((1, BM, dim), lambda j: (j // per_batch, j % per_batch, 0)),
            pl.BlockSpec((NUM_EXPERTS, dim), lambda j: (0, 0)),
            pl.BlockSpec((NUM_EXPERTS, 1), lambda j: (0, 0)),
        ],
        out_specs=[
            pl.BlockSpec((1, 1, BM), lambda j: (j, 0, 0)),
            pl.BlockSpec((1, 1, BM), lambda j: (j, 0, 0)),
            pl.BlockSpec((NUM_EXPERTS, 1), lambda j: (0, 0)),
        ],
        out_shape=[
            jax.ShapeDtypeStruct((nb, 1, BM), jnp.float32),
            jax.ShapeDtypeStruct((nb, 1, BM), jnp.int32),
            jax.ShapeDtypeStruct((NUM_EXPERTS, 1), jnp.float32),
        ],
    )(x, W, b2)

    expand = _make_sc_expand(batch, N)
    out_flat = expand(score.reshape(rows), amax.reshape(rows),
                      denom.reshape(NUM_EXPERTS))
    out_t = out_flat.reshape(batch, NUM_EXPERTS, N)
    return jnp.transpose(out_t, (0, 2, 1))


# speedup vs baseline: 1.6904x; 1.2149x over previous
import functools

import jax
import jax.numpy as jnp
from jax.experimental import pallas as pl
from jax.experimental.pallas import tpu as pltpu
from jax.experimental.pallas import tpu_sc as plsc

DIM = 768
NUM_EXPERTS = 64
CAPACITY_FACTOR = 1.0
EPSILON = 1e-06

BM = 4096
NC = 2
NS = 16


def _stage_a(x_ref, w_ref, b_ref, score_ref, amax_ref, denom_ref):
    j = pl.program_id(0)
    xb = x_ref[0]
    logits = jax.lax.dot_general(
        w_ref[...], xb,
        (((1,), (1,)), ((), ())),
        preferred_element_type=jnp.float32,
    ) + b_ref[...]
    m = jnp.max(logits, axis=0, keepdims=True)
    idx = jax.lax.broadcasted_iota(jnp.int32, logits.shape, 0)
    a = jnp.min(jnp.where(logits == m, idx, NUM_EXPERTS), axis=0)
    s = jnp.sum(jnp.exp(logits - m), axis=0)
    score = 1.0 / s
    score_ref[0, 0, :] = score
    amax_ref[0, 0, :] = a
    onehot = (idx == a[None, :]).astype(jnp.float32)
    contrib = jax.lax.dot_general(
        onehot * score[None, :], jnp.ones((BM, 1), jnp.float32),
        (((1,), (0,)), ((), ())),
        preferred_element_type=jnp.float32,
    )

    @pl.when(j == 0)
    def _():
        denom_ref[...] = jnp.zeros_like(denom_ref)

    denom_ref[...] += contrib


def _make_sc_expand(batch, n_tok):
    rows = batch * n_tok
    rpt = rows // (NC * NS)
    tiles_per_b = n_tok // rpt
    slab = NUM_EXPERTS * rpt
    cap = float(rows * CAPACITY_FACTOR)

    def body(score_hbm, amax_hbm, denom_hbm, out_ref,
             score_v, amax_v, denom_v, inv_v, buf, sem):
        wid = jax.lax.axis_index("s") * NC + jax.lax.axis_index("c")
        base = wid * rpt
        bidx = wid // tiles_per_b
        n0 = (wid % tiles_per_b) * rpt
        pltpu.sync_copy(score_hbm.at[pl.ds(base, rpt)], score_v)
        pltpu.sync_copy(amax_hbm.at[pl.ds(base, rpt)], amax_v)
        pltpu.sync_copy(denom_hbm, denom_v)
        for t in range(NUM_EXPERTS // 16):
            d = denom_v[pl.ds(t * 16, 16)]
            inv_v[pl.ds(t * 16, 16)] = cap / (d + EPSILON)

        zeros16 = jnp.zeros((16,), jnp.float32)

        def zb(e, carry):
            for u in range(rpt // 16):
                buf[e, pl.ds(u * 16, 16)] = zeros16
            return carry

        jax.lax.fori_loop(0, NUM_EXPERTS, zb, 0)

        iota16 = jax.lax.iota(jnp.int32, 16)
        for g in range(rpt // 16):
            rb = g * 16
            a16 = amax_v[pl.ds(rb, 16)]
            v16 = score_v[pl.ds(rb, 16)] * plsc.load_gather(inv_v, [a16])
            plsc.store_scatter(buf, [a16, rb + iota16], v16)

        pltpu.async_copy(
            buf, out_ref.at[bidx, :, pl.ds(n0, rpt)], sem
        ).wait()

    return functools.partial(
        pl.kernel,
        mesh=plsc.VectorSubcoreMesh(core_axis_name="c", subcore_axis_name="s"),
        compiler_params=pltpu.CompilerParams(needs_layout_passes=False),
        out_type=jax.ShapeDtypeStruct((batch, NUM_EXPERTS, n_tok), jnp.float32),
        scratch_types=[
            pltpu.VMEM((rpt,), jnp.float32),
            pltpu.VMEM((rpt,), jnp.int32),
            pltpu.VMEM((NUM_EXPERTS,), jnp.float32),
            pltpu.VMEM((NUM_EXPERTS,), jnp.float32),
            pltpu.VMEM((NUM_EXPERTS, rpt), jnp.float32),
            pltpu.SemaphoreType.DMA,
        ],
    )(body)


def kernel(x, W, b):
    batch, N, dim = x.shape
    rows = batch * N
    nb = rows // BM
    per_batch = N // BM
    b2 = b.reshape(NUM_EXPERTS, 1)

    score, amax, denom = pl.pallas_call(
        _stage_a,
        grid=(nb,),
        in_specs=[
            pl.BlockSpec((1, BM, dim), lambda j: (j // per_batch, j % per_batch, 0)),
            pl.BlockSpec((NUM_EXPERTS, dim), lambda j: (0, 0)),
            pl.BlockSpec((NUM_EXPERTS, 1), lambda j: (0, 0)),
        ],
        out_specs=[
            pl.BlockSpec((1, 1, BM), lambda j: (j, 0, 0)),
            pl.BlockSpec((1, 1, BM), lambda j: (j, 0, 0)),
            pl.BlockSpec((NUM_EXPERTS, 1), lambda j: (0, 0)),
        ],
        out_shape=[
            jax.ShapeDtypeStruct((nb, 1, BM), jnp.float32),
            jax.ShapeDtypeStruct((nb, 1, BM), jnp.int32),
            jax.ShapeDtypeStruct((NUM_EXPERTS, 1), jnp.float32),
        ],
    )(x, W, b2)

    expand = _make_sc_expand(batch, N)
    out_t = expand(score.reshape(rows), amax.reshape(rows),
                   denom.reshape(NUM_EXPERTS))
    return jnp.transpose(out_t, (0, 2, 1))


# speedup vs baseline: 1.7018x; 1.0068x over previous
import functools

import jax
import jax.numpy as jnp
from jax.experimental import pallas as pl
from jax.experimental.pallas import tpu as pltpu
from jax.experimental.pallas import tpu_sc as plsc

DIM = 768
NUM_EXPERTS = 64
CAPACITY_FACTOR = 1.0
EPSILON = 1e-06

BM = 4096
NC = 2
NS = 16


def _stage_a(x_ref, w_ref, b_ref, score_ref, amax_ref, denom_ref):
    j = pl.program_id(0)
    xb = x_ref[0]
    logits = jax.lax.dot_general(
        w_ref[...], xb,
        (((1,), (1,)), ((), ())),
        preferred_element_type=jnp.float32,
    ) + b_ref[...]
    m = jnp.max(logits, axis=0, keepdims=True)
    idx = jax.lax.broadcasted_iota(jnp.int32, logits.shape, 0)
    a = jnp.min(jnp.where(logits == m, idx, NUM_EXPERTS), axis=0)
    s = jnp.sum(jnp.exp(logits - m), axis=0)
    score = 1.0 / s
    score_ref[0, 0, :] = score
    amax_ref[0, 0, :] = a
    onehot = (idx == a[None, :]).astype(jnp.float32)
    contrib = jax.lax.dot_general(
        onehot * score[None, :], jnp.ones((BM, 1), jnp.float32),
        (((1,), (0,)), ((), ())),
        preferred_element_type=jnp.float32,
    )

    @pl.when(j == 0)
    def _():
        denom_ref[...] = jnp.zeros_like(denom_ref)

    denom_ref[...] += contrib


def _make_sc_expand(batch, n_tok):
    rows = batch * n_tok
    rpt = rows // (NC * NS)
    tiles_per_b = n_tok // rpt
    slab = NUM_EXPERTS * rpt
    cap = float(rows * CAPACITY_FACTOR)

    def body(score_hbm, amax_hbm, denom_hbm, out_ref,
             score_v, amax_v, denom_v, inv_v, buf, sem):
        wid = jax.lax.axis_index("s") * NC + jax.lax.axis_index("c")
        base = wid * rpt
        bidx = wid // tiles_per_b
        n0 = (wid % tiles_per_b) * rpt
        pltpu.sync_copy(score_hbm.at[pl.ds(base, rpt)], score_v)
        pltpu.sync_copy(amax_hbm.at[pl.ds(base, rpt)], amax_v)
        pltpu.sync_copy(denom_hbm, denom_v)
        for t in range(NUM_EXPERTS // 16):
            d = denom_v[pl.ds(t * 16, 16)]
            inv_v[pl.ds(t * 16, 16)] = cap / (d + EPSILON)

        zeros16 = jnp.zeros((16,), jnp.float32)

        def zb(e, carry):
            for u in range(rpt // 16):
                buf[e, pl.ds(u * 16, 16)] = zeros16
            return carry

        jax.lax.fori_loop(0, NUM_EXPERTS, zb, 0)

        iota16 = jax.lax.iota(jnp.int32, 16)

        def sb(g, carry):
            rb = g * 16
            a16 = amax_v[pl.ds(rb, 16)]
            v16 = score_v[pl.ds(rb, 16)] * plsc.load_gather(inv_v, [a16])
            plsc.store_scatter(buf, [a16, rb + iota16], v16)
            return carry

        jax.lax.fori_loop(0, rpt // 16, sb, 0)

        pltpu.async_copy(
            buf, out_ref.at[bidx, :, pl.ds(n0, rpt)], sem
        ).wait()

    return functools.partial(
        pl.kernel,
        mesh=plsc.VectorSubcoreMesh(core_axis_name="c", subcore_axis_name="s"),
        compiler_params=pltpu.CompilerParams(needs_layout_passes=False),
        out_type=jax.ShapeDtypeStruct((batch, NUM_EXPERTS, n_tok), jnp.float32),
        scratch_types=[
            pltpu.VMEM((rpt,), jnp.float32),
            pltpu.VMEM((rpt,), jnp.int32),
            pltpu.VMEM((NUM_EXPERTS,), jnp.float32),
            pltpu.VMEM((NUM_EXPERTS,), jnp.float32),
            pltpu.VMEM((NUM_EXPERTS, rpt), jnp.float32),
            pltpu.SemaphoreType.DMA,
        ],
    )(body)


def kernel(x, W, b):
    batch, N, dim = x.shape
    rows = batch * N
    nb = rows // BM
    per_batch = N // BM
    b2 = b.reshape(NUM_EXPERTS, 1)

    score, amax, denom = pl.pallas_call(
        _stage_a,
        grid=(nb,),
        in_specs=[
            pl.BlockSpec((1, BM, dim), lambda j: (j // per_batch, j % per_batch, 0)),
            pl.BlockSpec((NUM_EXPERTS, dim), lambda j: (0, 0)),
            pl.BlockSpec((NUM_EXPERTS, 1), lambda j: (0, 0)),
        ],
        out_specs=[
            pl.BlockSpec((1, 1, BM), lambda j: (j, 0, 0)),
            pl.BlockSpec((1, 1, BM), lambda j: (j, 0, 0)),
            pl.BlockSpec((NUM_EXPERTS, 1), lambda j: (0, 0)),
        ],
        out_shape=[
            jax.ShapeDtypeStruct((nb, 1, BM), jnp.float32),
            jax.ShapeDtypeStruct((nb, 1, BM), jnp.int32),
            jax.ShapeDtypeStruct((NUM_EXPERTS, 1), jnp.float32),
        ],
    )(x, W, b2)

    expand = _make_sc_expand(batch, N)
    out_t = expand(score.reshape(rows), amax.reshape(rows),
                   denom.reshape(NUM_EXPERTS))
    return jnp.transpose(out_t, (0, 2, 1))
